# baseline (speedup 1.0000x reference)
import dataclasses
import functools

import jax
import jax.numpy as jnp
from jax import lax
from jax.experimental import pallas as pl
from jax.experimental.pallas import tpu as pltpu
from jax.experimental.pallas import tpu_sc as plsc

B = 16384
F = 64
L = 16
NC = 2
NS = 16
NW = NC * NS
BPW = B // NW
GCH = 128
NCHUNK = BPW // GCH
GROUPS = BPW // L
NUSERS = 1000000
HALF = NUSERS // 2


def _mf_body(ulo_hbm, uhi_hbm, ubT_hbm, ie_hbm, ibT_hbm, uid_hbm, iid_hbm,
             out_hbm, uidx, lidx, hidx, iidx, ulo_rows, uhi_rows, i_rows,
             ubv, ibv, out_v, sem):
    wid = lax.axis_index("s") * NC + lax.axis_index("c")
    base = wid * BPW

    cps = []
    for k in range(NCHUNK):
        cps.append(pltpu.async_copy(
            uid_hbm.at[pl.ds(base + k * GCH, GCH)], uidx.at[k], sem))
        cps.append(pltpu.async_copy(
            iid_hbm.at[pl.ds(base + k * GCH, GCH)], iidx.at[k], sem))
    for c in cps:
        c.wait()

    for k in range(NCHUNK):
        for j in range(GCH // L):
            sl = pl.ds(j * L, L)
            v = uidx[k, sl]
            lidx[k, sl] = jnp.minimum(v, HALF - 1)
            hidx[k, sl] = jnp.clip(v - HALF, 0, HALF - 1)

    cps = []
    for k in range(NCHUNK):
        sl = pl.ds(k * GCH, GCH)
        cps.append(pltpu.async_copy(ulo_hbm.at[lidx.at[k]], ulo_rows.at[sl], sem))
        cps.append(pltpu.async_copy(uhi_hbm.at[hidx.at[k]], uhi_rows.at[sl], sem))
        cps.append(pltpu.async_copy(ie_hbm.at[iidx.at[k]], i_rows.at[sl], sem))
        cps.append(pltpu.async_copy(ubT_hbm.at[0].at[uidx.at[k]], ubv.at[sl], sem))
        cps.append(pltpu.async_copy(ibT_hbm.at[0].at[iidx.at[k]], ibv.at[sl], sem))
    for c in cps:
        c.wait()

    lane = lax.iota(jnp.int32, L)

    @pl.loop(0, GROUPS)
    def _(g):
        sl16 = pl.ds(g * L, L)
        rows = g * L + lane
        uid_v = plsc.load_gather(uidx, [rows >> 7, rows & 127])
        use_lo = uid_v < HALF
        acc = ubv[sl16] + ibv[sl16]
        for f in range(F):
            col = jnp.full((L,), f, jnp.int32)
            ulo = plsc.load_gather(ulo_rows, [rows, col])
            uhi = plsc.load_gather(uhi_rows, [rows, col])
            uv = jnp.where(use_lo, ulo, uhi)
            iv = plsc.load_gather(i_rows, [rows, col])
            acc = acc + uv * iv
        out_v[sl16] = 1.0 / (1.0 + jnp.exp(-acc))

    pltpu.sync_copy(out_v, out_hbm.at[pl.ds(base, BPW)])


@jax.jit
def _mf(user_embed, user_bias_embed, item_embed, item_bias_embed,
        user_ids, item_ids):
    cp = pltpu.CompilerParams()
    fields = pltpu.CompilerParams.__dataclass_fields__
    if "needs_layout_passes" in fields:
        cp = dataclasses.replace(cp, needs_layout_passes=False)
    if "use_tc_tiling_on_sc" in fields:
        cp = dataclasses.replace(cp, use_tc_tiling_on_sc=False)
    run = pl.kernel(
        _mf_body,
        out_type=jax.ShapeDtypeStruct((B,), jnp.float32),
        compiler_params=cp,
        mesh=plsc.VectorSubcoreMesh(core_axis_name="c", subcore_axis_name="s"),
        scratch_types=[
            pltpu.VMEM((NCHUNK, GCH), jnp.int32),
            pltpu.VMEM((NCHUNK, GCH), jnp.int32),
            pltpu.VMEM((NCHUNK, GCH), jnp.int32),
            pltpu.VMEM((NCHUNK, GCH), jnp.int32),
            pltpu.VMEM((BPW, F), jnp.float32),
            pltpu.VMEM((BPW, F), jnp.float32),
            pltpu.VMEM((BPW, F), jnp.float32),
            pltpu.VMEM((BPW,), jnp.float32),
            pltpu.VMEM((BPW,), jnp.float32),
            pltpu.VMEM((BPW,), jnp.float32),
            pltpu.SemaphoreType.DMA,
        ],
    )
    return run(user_embed[:HALF], user_embed[HALF:],
               user_bias_embed.T, item_embed, item_bias_embed.T,
               user_ids, item_ids)


def kernel(user_embed, user_bias_embed, item_embed, item_bias_embed,
           user_ids, item_ids):
    return _mf(user_embed, user_bias_embed, item_embed, item_bias_embed,
               user_ids.astype(jnp.int32), item_ids.astype(jnp.int32))


# speedup vs baseline: 3.1060x; 3.1060x over previous
import dataclasses
import functools

import jax
import jax.numpy as jnp
from jax import lax
from jax.experimental import pallas as pl
from jax.experimental.pallas import tpu as pltpu
from jax.experimental.pallas import tpu_sc as plsc

B = 16384
F = 64
L = 16
NC = 2
NS = 16
NW = NC * NS
BPW = B // NW
GCH = 128
NCHUNK = BPW // GCH
GROUPS = BPW // L
NBUF = 2


def _mf_body(ueT_hbm, ubT_hbm, ie2_hbm, ibT_hbm, uid_hbm, iid_hbm, out_hbm,
             uflat, uidx, iidx, hidx, tbuf, u_cols, i_pairs, ubv, ibv,
             out_v, gsem, tsem):
    wid = lax.axis_index("s") * NC + lax.axis_index("c")
    base = wid * BPW

    cps = [pltpu.async_copy(uid_hbm.at[pl.ds(base, BPW)], uflat, gsem)]
    for k in range(NCHUNK):
        cps.append(pltpu.async_copy(
            uid_hbm.at[pl.ds(base + k * GCH, GCH)], uidx.at[k], gsem))
        cps.append(pltpu.async_copy(
            iid_hbm.at[pl.ds(base + k * GCH, GCH)], iidx.at[k], gsem))
    for c in cps:
        c.wait()

    for k in range(NCHUNK):
        for j in range(GCH // L):
            sl = pl.ds(j * L, L)
            hidx[k, sl] = iidx[k, sl] >> 1

    cps = []
    for k in range(NCHUNK):
        sl = pl.ds(k * GCH, GCH)
        cps.append(pltpu.async_copy(ie2_hbm.at[hidx.at[k]], i_pairs.at[sl], gsem))
        cps.append(pltpu.async_copy(ubT_hbm.at[0].at[uidx.at[k]], ubv.at[sl], gsem))
        cps.append(pltpu.async_copy(ibT_hbm.at[0].at[iidx.at[k]], ibv.at[sl], gsem))

    lane = lax.iota(jnp.int32, L)
    frows = [q * L + lane for q in range(F // L)]

    def uid_at(j):
        vec = uflat[pl.ds((j >> 4) * L, L)]
        return jnp.sum(jnp.where(lane == (j & (L - 1)), vec, 0))

    def fire(j, slot):
        u = uid_at(j)
        toff = pl.multiple_of((u >> 7) * GCH, GCH)
        return pltpu.async_copy(ueT_hbm.at[:, pl.ds(toff, GCH)],
                                tbuf.at[slot], tsem)

    for b in range(NBUF):
        fire(b, b)

    @pl.loop(0, BPW, step=NBUF)
    def _(j0):
        for b in range(NBUF):
            j = j0 + b
            pltpu.make_async_copy(ueT_hbm.at[:, pl.ds(0, GCH)],
                                  tbuf.at[b], tsem).wait()
            u = uid_at(j)
            um = jnp.full((L,), u & (GCH - 1), jnp.int32)
            jcol = jnp.full((L,), j, jnp.int32)
            for q in range(F // L):
                vq = plsc.load_gather(tbuf.at[b], [frows[q], um])
                plsc.store_scatter(u_cols, [frows[q], jcol], vq)
            nj = j + NBUF

            @pl.when(nj < BPW)
            def _():
                fire(nj, b)

    for c in cps:
        c.wait()

    @pl.loop(0, GROUPS)
    def _(g):
        sl16 = pl.ds(g * L, L)
        rows = g * L + lane
        iid_v = plsc.load_gather(iidx, [rows >> 7, rows & 127])
        half = (iid_v & 1) * F
        acc = ubv[sl16] + ibv[sl16]
        for f in range(F):
            uv = u_cols[f, sl16]
            iv = plsc.load_gather(i_pairs, [rows, half + f])
            acc = acc + uv * iv
        out_v[sl16] = 1.0 / (1.0 + jnp.exp(-acc))

    pltpu.sync_copy(out_v, out_hbm.at[pl.ds(base, BPW)])


@jax.jit
def _mf(user_embed, user_bias_embed, item_embed, item_bias_embed,
        user_ids, item_ids):
    cp = pltpu.CompilerParams()
    fields = pltpu.CompilerParams.__dataclass_fields__
    if "needs_layout_passes" in fields:
        cp = dataclasses.replace(cp, needs_layout_passes=False)
    run = pl.kernel(
        _mf_body,
        out_type=jax.ShapeDtypeStruct((B,), jnp.float32),
        compiler_params=cp,
        mesh=plsc.VectorSubcoreMesh(core_axis_name="c", subcore_axis_name="s"),
        scratch_types=[
            pltpu.VMEM((BPW,), jnp.int32),
            pltpu.VMEM((NCHUNK, GCH), jnp.int32),
            pltpu.VMEM((NCHUNK, GCH), jnp.int32),
            pltpu.VMEM((NCHUNK, GCH), jnp.int32),
            pltpu.VMEM((NBUF, F, GCH), jnp.float32),
            pltpu.VMEM((F, BPW), jnp.float32),
            pltpu.VMEM((BPW, 2 * F), jnp.float32),
            pltpu.VMEM((BPW,), jnp.float32),
            pltpu.VMEM((BPW,), jnp.float32),
            pltpu.VMEM((BPW,), jnp.float32),
            pltpu.SemaphoreType.DMA,
            pltpu.SemaphoreType.DMA,
        ],
    )
    return run(user_embed.T, user_bias_embed.T,
               item_embed.reshape(-1, 2 * F), item_bias_embed.T,
               user_ids, item_ids)


def kernel(user_embed, user_bias_embed, item_embed, item_bias_embed,
           user_ids, item_ids):
    return _mf(user_embed, user_bias_embed, item_embed, item_bias_embed,
               user_ids.astype(jnp.int32), item_ids.astype(jnp.int32))


# speedup vs baseline: 4.0508x; 1.3042x over previous
import dataclasses
import functools

import jax
import jax.numpy as jnp
from jax import lax
from jax.experimental import pallas as pl
from jax.experimental.pallas import tpu as pltpu
from jax.experimental.pallas import tpu_sc as plsc

B = 16384
F = 64
L = 16
NC = 2
NS = 16
NW = NC * NS
BPW = B // NW
GCH = 128
NCHUNK = BPW // GCH
GROUPS = BPW // L
NBUF = 4


def _mf_body(ueT_hbm, ubT_hbm, ie2_hbm, ibT_hbm, uid_hbm, iid_hbm, out_hbm,
             uflat, uidx, iidx, hidx, tbuf, u_cols, ip_chunk, i_cols,
             ubv, ibv, out_v, gsem, isem, tsem):
    wid = lax.axis_index("s") * NC + lax.axis_index("c")
    base = wid * BPW

    cps = [pltpu.async_copy(uid_hbm.at[pl.ds(base, BPW)], uflat, gsem)]
    for k in range(NCHUNK):
        cps.append(pltpu.async_copy(
            uid_hbm.at[pl.ds(base + k * GCH, GCH)], uidx.at[k], gsem))
        cps.append(pltpu.async_copy(
            iid_hbm.at[pl.ds(base + k * GCH, GCH)], iidx.at[k], gsem))
    for c in cps:
        c.wait()

    for k in range(NCHUNK):
        for j in range(GCH // L):
            sl = pl.ds(j * L, L)
            hidx[k, sl] = iidx[k, sl] >> 1

    cps = []
    for k in range(NCHUNK):
        sl = pl.ds(k * GCH, GCH)
        cps.append(pltpu.async_copy(ubT_hbm.at[0].at[uidx.at[k]], ubv.at[sl], gsem))
        cps.append(pltpu.async_copy(ibT_hbm.at[0].at[iidx.at[k]], ibv.at[sl], gsem))

    lane = lax.iota(jnp.int32, L)
    frows = [q * L + lane for q in range(F // L)]

    ic = pltpu.async_copy(ie2_hbm.at[hidx.at[0]], ip_chunk, isem)
    for k in range(NCHUNK):
        ic.wait()

        @pl.loop(0, GCH // L)
        def _(g2):
            rows16 = g2 * L + lane
            iid_v = plsc.load_gather(iidx, [jnp.full((L,), k, jnp.int32), rows16])
            half = (iid_v & 1) * F
            jcol = k * GCH + rows16
            for f in range(F):
                val = plsc.load_gather(ip_chunk, [rows16, half + f])
                plsc.store_scatter(i_cols, [jnp.full((L,), f, jnp.int32), jcol], val)

        if k + 1 < NCHUNK:
            ic = pltpu.async_copy(ie2_hbm.at[hidx.at[k + 1]], ip_chunk, isem)

    def uid_at(j):
        vec = uflat[pl.ds((j >> 4) * L, L)]
        return jnp.sum(jnp.where(lane == (j & (L - 1)), vec, 0))

    def fire(j, slot):
        u = uid_at(j)
        toff = pl.multiple_of((u >> 7) * GCH, GCH)
        return pltpu.async_copy(ueT_hbm.at[:, pl.ds(toff, GCH)],
                                tbuf.at[slot], tsem)

    for b in range(NBUF):
        fire(b, b)

    @pl.loop(0, BPW, step=NBUF)
    def _(j0):
        for b in range(NBUF):
            j = j0 + b
            pltpu.make_async_copy(ueT_hbm.at[:, pl.ds(0, GCH)],
                                  tbuf.at[b], tsem).wait()
            u = uid_at(j)
            um = jnp.full((L,), u & (GCH - 1), jnp.int32)
            jcol = jnp.full((L,), j, jnp.int32)
            for q in range(F // L):
                vq = plsc.load_gather(tbuf.at[b], [frows[q], um])
                plsc.store_scatter(u_cols, [frows[q], jcol], vq)
            nj = j + NBUF

            @pl.when(nj < BPW)
            def _():
                fire(nj, b)

    for c in cps:
        c.wait()

    @pl.loop(0, GROUPS)
    def _(g):
        sl16 = pl.ds(g * L, L)
        acc = ubv[sl16] + ibv[sl16]
        for f in range(F):
            acc = acc + u_cols[f, sl16] * i_cols[f, sl16]
        out_v[sl16] = 1.0 / (1.0 + jnp.exp(-acc))

    pltpu.sync_copy(out_v, out_hbm.at[pl.ds(base, BPW)])


@jax.jit
def _mf(user_embed, user_bias_embed, item_embed, item_bias_embed,
        user_ids, item_ids):
    cp = pltpu.CompilerParams()
    fields = pltpu.CompilerParams.__dataclass_fields__
    if "needs_layout_passes" in fields:
        cp = dataclasses.replace(cp, needs_layout_passes=False)
    run = pl.kernel(
        _mf_body,
        out_type=jax.ShapeDtypeStruct((B,), jnp.float32),
        compiler_params=cp,
        mesh=plsc.VectorSubcoreMesh(core_axis_name="c", subcore_axis_name="s"),
        scratch_types=[
            pltpu.VMEM((BPW,), jnp.int32),
            pltpu.VMEM((NCHUNK, GCH), jnp.int32),
            pltpu.VMEM((NCHUNK, GCH), jnp.int32),
            pltpu.VMEM((NCHUNK, GCH), jnp.int32),
            pltpu.VMEM((NBUF, F, GCH), jnp.float32),
            pltpu.VMEM((F, BPW), jnp.float32),
            pltpu.VMEM((GCH, 2 * F), jnp.float32),
            pltpu.VMEM((F, BPW), jnp.float32),
            pltpu.VMEM((BPW,), jnp.float32),
            pltpu.VMEM((BPW,), jnp.float32),
            pltpu.VMEM((BPW,), jnp.float32),
            pltpu.SemaphoreType.DMA,
            pltpu.SemaphoreType.DMA,
            pltpu.SemaphoreType.DMA,
        ],
    )
    return run(user_embed.T, user_bias_embed.T,
               item_embed.reshape(-1, 2 * F), item_bias_embed.T,
               user_ids, item_ids)


def kernel(user_embed, user_bias_embed, item_embed, item_bias_embed,
           user_ids, item_ids):
    return _mf(user_embed, user_bias_embed, item_embed, item_bias_embed,
               user_ids.astype(jnp.int32), item_ids.astype(jnp.int32))
